# baseline (speedup 1.0000x reference)
import jax
import jax.numpy as jnp
from jax import lax
from jax.experimental import pallas as pl
from jax.experimental.pallas import tpu as pltpu
from jax.experimental.pallas import tpu_sc as plsc

N = 10000
E = 320000
D = 128
NP = 10240
NC = 2
NS = 16
NW = NC * NS
EPW = E // NW
CS = 128
CH = NP // CS
RPT = NP // NS
KO = RPT // CS
PAD_SRC = N
PAD_DST = NP - 1

_mesh = plsc.VectorSubcoreMesh(core_axis_name="c", subcore_axis_name="s",
                               num_cores=NC, num_subcores=NS)


def _row_fill(ref, n, valfn):
    def f(i, _):
        for j16 in range(D // 16):
            ref[i, pl.ds(j16 * 16, 16)] = valfn(i, j16)
        return 0
    lax.fori_loop(0, n, f, 0)


def _fill_own_idx(idx_ref, s, k):
    base = s * RPT + k * CS
    for j16 in range(CS // 16):
        idx_ref[pl.ds(j16 * 16, 16)] = (base + j16 * 16
                                        + lax.iota(jnp.int32, 16))


def _edge_body(hs_hbm, src_hbm, dst_hbm, agg_out, rows0, rows1,
               sidx0, didx0, sidx1, didx1, acc, g0, g1, s0, s1):
    c = lax.axis_index("c")
    s = lax.axis_index("s")
    w = c * NS + s
    _row_fill(rows0, CS, lambda i, j16: jnp.zeros((16,), jnp.float32))
    for k in range(KO):
        _fill_own_idx(didx0, s, k)
        pltpu.async_copy(rows0, acc.at[didx0], s0).wait()
    plsc.subcore_barrier()

    def body(i, _):
        j = 2 * i
        pltpu.sync_copy(src_hbm.at[w, j], sidx0)
        pltpu.sync_copy(dst_hbm.at[w, j], didx0)
        pltpu.sync_copy(src_hbm.at[w, j + 1], sidx1)
        pltpu.sync_copy(dst_hbm.at[w, j + 1], didx1)
        cg0 = pltpu.async_copy(hs_hbm.at[sidx0], rows0, g0)
        cg1 = pltpu.async_copy(hs_hbm.at[sidx1], rows1, g1)
        cg0.wait()
        cs0 = pltpu.async_copy(rows0, acc.at[didx0], s0, add=True)
        cg1.wait()
        cs1 = pltpu.async_copy(rows1, acc.at[didx1], s1, add=True)
        cs0.wait()
        cs1.wait()
        return 0
    lax.fori_loop(0, CH // 2, body, 0)

    plsc.subcore_barrier()
    for k in range(KO):
        _fill_own_idx(didx0, s, k)
        pltpu.async_copy(acc.at[didx0], rows0, g0).wait()
        pltpu.sync_copy(rows0, agg_out.at[c, pl.ds(s * RPT + k * CS, CS)])


_edge_kernel = pl.kernel(
    _edge_body,
    out_type=jax.ShapeDtypeStruct((NC, NP, D), jnp.float32),
    mesh=_mesh,
    scratch_types=[
        pltpu.VMEM((CS, D), jnp.float32),
        pltpu.VMEM((CS, D), jnp.float32),
        pltpu.VMEM((CS,), jnp.int32),
        pltpu.VMEM((CS,), jnp.int32),
        pltpu.VMEM((CS,), jnp.int32),
        pltpu.VMEM((CS,), jnp.int32),
        pltpu.VMEM_SHARED((NP, D), jnp.float32),
        pltpu.SemaphoreType.DMA,
        pltpu.SemaphoreType.DMA,
        pltpu.SemaphoreType.DMA,
        pltpu.SemaphoreType.DMA,
    ],
)


def _dis_from_deg(deg_blk):
    counts = deg_blk[0, :, 0] + deg_blk[1, :, 0] + 1.0
    return lax.rsqrt(counts)


def _b1_body(x_ref, w_ref, deg_ref, out_ref):
    dis = _dis_from_deg(deg_ref[...])
    h = jnp.dot(x_ref[...], w_ref[...], preferred_element_type=jnp.float32)
    out_ref[...] = h * dis[:, None]


def _bn_body(agg_ref, hs_ref, deg_ref, b_ref, w_ref, out_ref):
    dis = _dis_from_deg(deg_ref[...])
    pre = dis[:, None] * (agg_ref[0] + agg_ref[1] + hs_ref[...]) + b_ref[...]
    h = jnp.maximum(pre, 0.0)
    h = jnp.dot(h, w_ref[...], preferred_element_type=jnp.float32)
    out_ref[...] = h * dis[:, None]


def _fin_body(agg_ref, hs_ref, deg_ref, b_ref, wl_ref, bl_ref, out_ref):
    dis = _dis_from_deg(deg_ref[...])
    pre = dis[:, None] * (agg_ref[0] + agg_ref[1] + hs_ref[...]) + b_ref[...]
    h = jnp.maximum(pre, 0.0)
    out_ref[...] = (
        jnp.dot(h, wl_ref[...], preferred_element_type=jnp.float32)
        + bl_ref[...]
    )


_BLK = 128
_GRID = NP // _BLK

_spec_rows = pl.BlockSpec((_BLK, D), lambda i: (i, 0))
_spec_w = pl.BlockSpec((D, D), lambda i: (0, 0))
_spec_agg = pl.BlockSpec((NC, _BLK, D), lambda i: (0, i, 0))
_spec_b = pl.BlockSpec((1, D), lambda i: (0, 0))

_b1_call = pl.pallas_call(
    _b1_body,
    grid=(_GRID,),
    in_specs=[_spec_rows, _spec_w, _spec_agg],
    out_specs=_spec_rows,
    out_shape=jax.ShapeDtypeStruct((NP, D), jnp.float32),
)

_bn_call = pl.pallas_call(
    _bn_body,
    grid=(_GRID,),
    in_specs=[_spec_agg, _spec_rows, _spec_agg, _spec_b, _spec_w],
    out_specs=_spec_rows,
    out_shape=jax.ShapeDtypeStruct((NP, D), jnp.float32),
)

_fin_call = pl.pallas_call(
    _fin_body,
    grid=(1,),
    in_specs=[
        pl.BlockSpec((NC, 8, D), lambda i: (0, 0, 0)),
        pl.BlockSpec((8, D), lambda i: (0, 0)),
        pl.BlockSpec((NC, 8, D), lambda i: (0, 0, 0)),
        _spec_b,
        _spec_w,
        _spec_b,
    ],
    out_specs=pl.BlockSpec((8, D), lambda i: (0, 0)),
    out_shape=jax.ShapeDtypeStruct((8, D), jnp.float32),
)


def kernel(x, edge_index, W1, b1, W2, b2, W3, b3, Wl, bl):
    x_pad = jnp.pad(x, ((0, NP - N), (0, 0)))
    src = edge_index[0].reshape(NW, EPW)
    dst = edge_index[1].reshape(NW, EPW)
    pad = NP - EPW
    src_pad = jnp.pad(src, ((0, 0), (0, pad)),
                      constant_values=PAD_SRC).reshape(NW, CH, CS)
    dst_pad = jnp.pad(dst, ((0, 0), (0, pad)),
                      constant_values=PAD_DST).reshape(NW, CH, CS)
    b1r = b1.reshape(1, D)
    b2r = b2.reshape(1, D)
    b3r = b3.reshape(1, D)
    wl_pad = jnp.pad(Wl, ((0, 0), (0, D - Wl.shape[1])))
    bl_pad = jnp.pad(bl, ((0, D - bl.shape[0]),)).reshape(1, D)

    ones_table = jnp.ones((NP, D), jnp.float32)
    src_deg = jnp.zeros_like(src_pad)
    deg = _edge_kernel(ones_table, src_deg, dst_pad)

    hs1 = _b1_call(x_pad, W1, deg)
    agg1 = _edge_kernel(hs1, src_pad, dst_pad)
    hs2 = _bn_call(agg1, hs1, deg, b1r, W2)
    agg2 = _edge_kernel(hs2, src_pad, dst_pad)
    hs3 = _bn_call(agg2, hs2, deg, b2r, W3)
    agg3 = _edge_kernel(hs3, src_pad, dst_pad)
    out = _fin_call(agg3, hs3, deg, b3r, wl_pad, bl_pad)
    return out[:5, :3]


# speedup vs baseline: 1.0084x; 1.0084x over previous
import jax
import jax.numpy as jnp
from jax import lax
from jax.experimental import pallas as pl
from jax.experimental.pallas import tpu as pltpu
from jax.experimental.pallas import tpu_sc as plsc

N = 10000
E = 320000
D = 128
NP = 10240
NC = 2
NS = 16
NW = NC * NS
EPW = E // NW
CS = 128
CH = NP // CS
RPT = NP // NS
KO = RPT // CS
PAD_SRC = N
PAD_DST = NP - 1

_mesh = plsc.VectorSubcoreMesh(core_axis_name="c", subcore_axis_name="s",
                               num_cores=NC, num_subcores=NS)


def _row_fill(ref, n, valfn):
    def f(i, _):
        for j16 in range(D // 16):
            ref[i, pl.ds(j16 * 16, 16)] = valfn(i, j16)
        return 0
    lax.fori_loop(0, n, f, 0)


def _fill_own_idx(idx_ref, s, k):
    base = s * RPT + k * CS
    for j16 in range(CS // 16):
        idx_ref[pl.ds(j16 * 16, 16)] = (base + j16 * 16
                                        + lax.iota(jnp.int32, 16))


_BC = 10
_NB = CH // _BC


def _edge_body(hs_hbm, idx_hbm, agg_out, rows0, ibuf, didx, acc, g0, s0):
    c = lax.axis_index("c")
    s = lax.axis_index("s")
    w = c * NS + s
    _row_fill(rows0, CS, lambda i, j16: jnp.zeros((16,), jnp.float32))
    for k in range(KO):
        _fill_own_idx(didx, s, k)
        pltpu.async_copy(rows0, acc.at[didx], s0).wait()
    plsc.subcore_barrier()

    def body(b, _):
        pltpu.sync_copy(idx_hbm.at[w, b], ibuf)
        for k in range(_BC):
            pltpu.async_copy(hs_hbm.at[ibuf.at[2 * k]], rows0, g0).wait()
            pltpu.async_copy(rows0, acc.at[ibuf.at[2 * k + 1]], s0,
                             add=True).wait()
        return 0
    lax.fori_loop(0, _NB, body, 0)

    plsc.subcore_barrier()
    for k in range(KO):
        _fill_own_idx(didx, s, k)
        pltpu.async_copy(acc.at[didx], rows0, g0).wait()
        pltpu.sync_copy(rows0, agg_out.at[c, pl.ds(s * RPT + k * CS, CS)])


_edge_kernel = pl.kernel(
    _edge_body,
    out_type=jax.ShapeDtypeStruct((NC, NP, D), jnp.float32),
    mesh=_mesh,
    scratch_types=[
        pltpu.VMEM((CS, D), jnp.float32),
        pltpu.VMEM((2 * _BC, CS), jnp.int32),
        pltpu.VMEM((CS,), jnp.int32),
        pltpu.VMEM_SHARED((NP, D), jnp.float32),
        pltpu.SemaphoreType.DMA,
        pltpu.SemaphoreType.DMA,
    ],
)


def _dis_from_deg(deg_blk):
    counts = deg_blk[0, :, 0] + deg_blk[1, :, 0] + 1.0
    return lax.rsqrt(counts)


def _b1_body(x_ref, w_ref, deg_ref, out_ref):
    dis = _dis_from_deg(deg_ref[...])
    h = jnp.dot(x_ref[...], w_ref[...], preferred_element_type=jnp.float32)
    out_ref[...] = h * dis[:, None]


def _bn_body(agg_ref, hs_ref, deg_ref, b_ref, w_ref, out_ref):
    dis = _dis_from_deg(deg_ref[...])
    pre = dis[:, None] * (agg_ref[0] + agg_ref[1] + hs_ref[...]) + b_ref[...]
    h = jnp.maximum(pre, 0.0)
    h = jnp.dot(h, w_ref[...], preferred_element_type=jnp.float32)
    out_ref[...] = h * dis[:, None]


def _fin_body(agg_ref, hs_ref, deg_ref, b_ref, wl_ref, bl_ref, out_ref):
    dis = _dis_from_deg(deg_ref[...])
    pre = dis[:, None] * (agg_ref[0] + agg_ref[1] + hs_ref[...]) + b_ref[...]
    h = jnp.maximum(pre, 0.0)
    out_ref[...] = (
        jnp.dot(h, wl_ref[...], preferred_element_type=jnp.float32)
        + bl_ref[...]
    )


_BLK = 128
_GRID = NP // _BLK

_spec_rows = pl.BlockSpec((_BLK, D), lambda i: (i, 0))
_spec_w = pl.BlockSpec((D, D), lambda i: (0, 0))
_spec_agg = pl.BlockSpec((NC, _BLK, D), lambda i: (0, i, 0))
_spec_b = pl.BlockSpec((1, D), lambda i: (0, 0))

_b1_call = pl.pallas_call(
    _b1_body,
    grid=(_GRID,),
    in_specs=[_spec_rows, _spec_w, _spec_agg],
    out_specs=_spec_rows,
    out_shape=jax.ShapeDtypeStruct((NP, D), jnp.float32),
)

_bn_call = pl.pallas_call(
    _bn_body,
    grid=(_GRID,),
    in_specs=[_spec_agg, _spec_rows, _spec_agg, _spec_b, _spec_w],
    out_specs=_spec_rows,
    out_shape=jax.ShapeDtypeStruct((NP, D), jnp.float32),
)

_fin_call = pl.pallas_call(
    _fin_body,
    grid=(1,),
    in_specs=[
        pl.BlockSpec((NC, 8, D), lambda i: (0, 0, 0)),
        pl.BlockSpec((8, D), lambda i: (0, 0)),
        pl.BlockSpec((NC, 8, D), lambda i: (0, 0, 0)),
        _spec_b,
        _spec_w,
        _spec_b,
    ],
    out_specs=pl.BlockSpec((8, D), lambda i: (0, 0)),
    out_shape=jax.ShapeDtypeStruct((8, D), jnp.float32),
)


def kernel(x, edge_index, W1, b1, W2, b2, W3, b3, Wl, bl):
    x_pad = jnp.pad(x, ((0, NP - N), (0, 0)))
    src = edge_index[0].reshape(NW, EPW)
    dst = edge_index[1].reshape(NW, EPW)
    pad = NP - EPW
    src_pad = jnp.pad(src, ((0, 0), (0, pad)),
                      constant_values=PAD_SRC).reshape(NW, CH, CS)
    dst_pad = jnp.pad(dst, ((0, 0), (0, pad)),
                      constant_values=PAD_DST).reshape(NW, CH, CS)
    comb = jnp.stack([src_pad, dst_pad], axis=2)
    comb = comb.reshape(NW, _NB, _BC * 2, CS)
    b1r = b1.reshape(1, D)
    b2r = b2.reshape(1, D)
    b3r = b3.reshape(1, D)
    wl_pad = jnp.pad(Wl, ((0, 0), (0, D - Wl.shape[1])))
    bl_pad = jnp.pad(bl, ((0, D - bl.shape[0]),)).reshape(1, D)

    ones_table = jnp.ones((NP, D), jnp.float32)
    comb_deg = comb.at[:, :, ::2].set(0)
    deg = _edge_kernel(ones_table, comb_deg)

    hs1 = _b1_call(x_pad, W1, deg)
    agg1 = _edge_kernel(hs1, comb)
    hs2 = _bn_call(agg1, hs1, deg, b1r, W2)
    agg2 = _edge_kernel(hs2, comb)
    hs3 = _bn_call(agg2, hs2, deg, b2r, W3)
    agg3 = _edge_kernel(hs3, comb)
    out = _fin_call(agg3, hs3, deg, b3r, wl_pad, bl_pad)
    return out[:5, :3]


# speedup vs baseline: 6.9603x; 6.9025x over previous
import jax
import jax.numpy as jnp
from jax import lax
from jax.experimental import pallas as pl
from jax.experimental.pallas import tpu as pltpu
from jax.experimental.pallas import tpu_sc as plsc

N = 10000
E = 320000
D = 128
NP = 10240
NC = 2
NS = 16
NW = NC * NS
EPW = E // NW
CS = 128
CH = NP // CS
RPT = NP // NS
KO = RPT // CS
PAD_SRC = N
PAD_DST = NP - 1

_mesh = plsc.VectorSubcoreMesh(core_axis_name="c", subcore_axis_name="s",
                               num_cores=NC, num_subcores=NS)


def _row_fill(ref, n, valfn):
    def f(i, _):
        for j16 in range(D // 16):
            ref[i, pl.ds(j16 * 16, 16)] = valfn(i, j16)
        return 0
    lax.fori_loop(0, n, f, 0)


def _fill_own_idx(idx_ref, s, k):
    base = s * RPT + k * CS
    for j16 in range(CS // 16):
        idx_ref[pl.ds(j16 * 16, 16)] = (base + j16 * 16
                                        + lax.iota(jnp.int32, 16))


_BC = 10
_NB = CH // _BC


def _edge_body(hs_hbm, idx_hbm, agg_out, rows0, ibuf, didx, acc, g0, s0):
    c = lax.axis_index("c")
    s = lax.axis_index("s")
    w = c * NS + s
    _row_fill(rows0, CS, lambda i, j16: jnp.zeros((16,), jnp.float32))
    for k in range(KO):
        _fill_own_idx(didx, s, k)
        pltpu.async_copy(rows0, acc.at[didx], s0).wait()
    plsc.subcore_barrier()

    def body(b, _):
        pltpu.sync_copy(idx_hbm.at[w, b], ibuf)
        for k in range(_BC):
            pltpu.async_copy(hs_hbm.at[ibuf.at[2 * k]], rows0, g0).wait()
            pltpu.async_copy(rows0, acc.at[ibuf.at[2 * k + 1]], s0,
                             add=True).wait()
        return 0
    lax.fori_loop(0, _NB, body, 0)

    plsc.subcore_barrier()
    for k in range(KO):
        _fill_own_idx(didx, s, k)
        pltpu.async_copy(acc.at[didx], rows0, g0).wait()
        pltpu.sync_copy(rows0, agg_out.at[c, pl.ds(s * RPT + k * CS, CS)])


_edge_kernel = pl.kernel(
    _edge_body,
    out_type=jax.ShapeDtypeStruct((NC, NP, D), jnp.float32),
    mesh=_mesh,
    scratch_types=[
        pltpu.VMEM((CS, D), jnp.float32),
        pltpu.VMEM((2 * _BC, CS), jnp.int32),
        pltpu.VMEM((CS,), jnp.int32),
        pltpu.VMEM_SHARED((NP, D), jnp.float32),
        pltpu.SemaphoreType.DMA,
        pltpu.SemaphoreType.DMA,
    ],
)


def _dis_from_deg(deg_blk):
    counts = deg_blk[0, :, 0] + deg_blk[1, :, 0] + 1.0
    return lax.rsqrt(counts)


def _b1_body(x_ref, w_ref, deg_ref, out_ref):
    dis = _dis_from_deg(deg_ref[...])
    h = jnp.dot(x_ref[...], w_ref[...], preferred_element_type=jnp.float32)
    out_ref[...] = h * dis[:, None]


def _bn_body(agg_ref, hs_ref, deg_ref, b_ref, w_ref, out_ref):
    dis = _dis_from_deg(deg_ref[...])
    pre = dis[:, None] * (agg_ref[0] + agg_ref[1] + hs_ref[...]) + b_ref[...]
    h = jnp.maximum(pre, 0.0)
    h = jnp.dot(h, w_ref[...], preferred_element_type=jnp.float32)
    out_ref[...] = h * dis[:, None]


def _fin_body(agg_ref, hs_ref, deg_ref, b_ref, wl_ref, bl_ref, out_ref):
    dis = _dis_from_deg(deg_ref[...])
    pre = dis[:, None] * (agg_ref[0] + agg_ref[1] + hs_ref[...]) + b_ref[...]
    h = jnp.maximum(pre, 0.0)
    out_ref[...] = (
        jnp.dot(h, wl_ref[...], preferred_element_type=jnp.float32)
        + bl_ref[...]
    )


_BLK = 128
_GRID = NP // _BLK

_spec_rows = pl.BlockSpec((_BLK, D), lambda i: (i, 0))
_spec_w = pl.BlockSpec((D, D), lambda i: (0, 0))
_spec_agg = pl.BlockSpec((NC, _BLK, D), lambda i: (0, i, 0))
_spec_b = pl.BlockSpec((1, D), lambda i: (0, 0))

_b1_call = pl.pallas_call(
    _b1_body,
    grid=(_GRID,),
    in_specs=[_spec_rows, _spec_w, _spec_agg],
    out_specs=_spec_rows,
    out_shape=jax.ShapeDtypeStruct((NP, D), jnp.float32),
)

_bn_call = pl.pallas_call(
    _bn_body,
    grid=(_GRID,),
    in_specs=[_spec_agg, _spec_rows, _spec_agg, _spec_b, _spec_w],
    out_specs=_spec_rows,
    out_shape=jax.ShapeDtypeStruct((NP, D), jnp.float32),
)

_fin_call = pl.pallas_call(
    _fin_body,
    grid=(1,),
    in_specs=[
        pl.BlockSpec((NC, 8, D), lambda i: (0, 0, 0)),
        pl.BlockSpec((8, D), lambda i: (0, 0)),
        pl.BlockSpec((NC, 8, D), lambda i: (0, 0, 0)),
        _spec_b,
        _spec_w,
        _spec_b,
    ],
    out_specs=pl.BlockSpec((8, D), lambda i: (0, 0)),
    out_shape=jax.ShapeDtypeStruct((8, D), jnp.float32),
)


def kernel(x, edge_index, W1, b1, W2, b2, W3, b3, Wl, bl):
    x_pad = jnp.pad(x, ((0, NP - N), (0, 0)))
    src = edge_index[0].reshape(NW, EPW)
    dst = edge_index[1].reshape(NW, EPW)
    pad = NP - EPW
    src_pad = jnp.pad(src, ((0, 0), (0, pad)),
                      constant_values=PAD_SRC).reshape(NW, CH, CS)
    dst_pad = jnp.pad(dst, ((0, 0), (0, pad)),
                      constant_values=PAD_DST).reshape(NW, CH, CS)
    comb = jnp.stack([src_pad, dst_pad], axis=2)
    comb = comb.reshape(NW, _NB, _BC * 2, CS)
    b1r = b1.reshape(1, D)
    b2r = b2.reshape(1, D)
    b3r = b3.reshape(1, D)
    wl_pad = jnp.pad(Wl, ((0, 0), (0, D - Wl.shape[1])))
    bl_pad = jnp.pad(bl, ((0, D - bl.shape[0]),)).reshape(1, D)

    ones_table = jnp.ones((NP, D), jnp.float32)
    deg = _edge_kernel(ones_table, comb)

    hs1 = _b1_call(x_pad, W1, deg)
    agg1 = _edge_kernel(hs1, comb)
    hs2 = _bn_call(agg1, hs1, deg, b1r, W2)
    agg2 = _edge_kernel(hs2, comb)
    hs3 = _bn_call(agg2, hs2, deg, b2r, W3)
    agg3 = _edge_kernel(hs3, comb)
    out = _fin_call(agg3, hs3, deg, b3r, wl_pad, bl_pad)
    return out[:5, :3]


# speedup vs baseline: 7.5428x; 1.0837x over previous
import jax
import jax.numpy as jnp
from jax import lax
from jax.experimental import pallas as pl
from jax.experimental.pallas import tpu as pltpu
from jax.experimental.pallas import tpu_sc as plsc

N = 10000
E = 320000
D = 128
NP = 10240
NC = 2
NS = 16
NW = NC * NS
EPW = E // NW
CS = 128
CH = NP // CS
RPT = NP // NS
KO = RPT // CS
PAD_SRC = N
PAD_DST = NP - 1

_mesh = plsc.VectorSubcoreMesh(core_axis_name="c", subcore_axis_name="s",
                               num_cores=NC, num_subcores=NS)


def _row_fill(ref, n, valfn):
    def f(i, _):
        for j16 in range(D // 16):
            ref[i, pl.ds(j16 * 16, 16)] = valfn(i, j16)
        return 0
    lax.fori_loop(0, n, f, 0)


def _fill_own_idx(idx_ref, s, k):
    base = s * RPT + k * CS
    for j16 in range(CS // 16):
        idx_ref[pl.ds(j16 * 16, 16)] = (base + j16 * 16
                                        + lax.iota(jnp.int32, 16))


_BC = 10
_NB = CH // _BC


def _edge_body(hs_hbm, idx_hbm, agg_out, rows0, rows1, ibuf, didx, acc,
               g0, g1, s0, s1):
    c = lax.axis_index("c")
    s = lax.axis_index("s")
    w = c * NS + s
    _row_fill(rows0, CS, lambda i, j16: jnp.zeros((16,), jnp.float32))
    for k in range(KO):
        _fill_own_idx(didx, s, k)
        pltpu.async_copy(rows0, acc.at[didx], s0).wait()
    plsc.subcore_barrier()

    bufs = (rows0, rows1)
    gsems = (g0, g1)
    ssems = (s0, s1)

    def body(b, _):
        pltpu.sync_copy(idx_hbm.at[w, b], ibuf)
        cg = pltpu.async_copy(hs_hbm.at[ibuf.at[0]], rows0, g0)
        cs_pair = [None, None]
        for k in range(_BC):
            p = k & 1
            cg.wait()
            cs_pair[p] = pltpu.async_copy(
                bufs[p], acc.at[ibuf.at[2 * k + 1]], ssems[p], add=True)
            if k + 1 < _BC:
                if cs_pair[1 - p] is not None:
                    cs_pair[1 - p].wait()
                cg = pltpu.async_copy(
                    hs_hbm.at[ibuf.at[2 * k + 2]], bufs[1 - p],
                    gsems[1 - p])
        cs_pair[0].wait()
        cs_pair[1].wait()
        return 0
    lax.fori_loop(0, _NB, body, 0)

    plsc.subcore_barrier()
    for k in range(KO):
        _fill_own_idx(didx, s, k)
        pltpu.async_copy(acc.at[didx], rows0, g0).wait()
        pltpu.sync_copy(rows0, agg_out.at[c, pl.ds(s * RPT + k * CS, CS)])


_edge_kernel = pl.kernel(
    _edge_body,
    out_type=jax.ShapeDtypeStruct((NC, NP, D), jnp.float32),
    mesh=_mesh,
    scratch_types=[
        pltpu.VMEM((CS, D), jnp.float32),
        pltpu.VMEM((CS, D), jnp.float32),
        pltpu.VMEM((2 * _BC, CS), jnp.int32),
        pltpu.VMEM((CS,), jnp.int32),
        pltpu.VMEM_SHARED((NP, D), jnp.float32),
        pltpu.SemaphoreType.DMA,
        pltpu.SemaphoreType.DMA,
        pltpu.SemaphoreType.DMA,
        pltpu.SemaphoreType.DMA,
    ],
)


def _dis_from_deg(deg_blk):
    counts = deg_blk[0, :, 0] + deg_blk[1, :, 0] + 1.0
    return lax.rsqrt(counts)


def _b1_body(x_ref, w_ref, deg_ref, out_ref):
    dis = _dis_from_deg(deg_ref[...])
    h = jnp.dot(x_ref[...], w_ref[...], preferred_element_type=jnp.float32)
    out_ref[...] = h * dis[:, None]


def _bn_body(agg_ref, hs_ref, deg_ref, b_ref, w_ref, out_ref):
    dis = _dis_from_deg(deg_ref[...])
    pre = dis[:, None] * (agg_ref[0] + agg_ref[1] + hs_ref[...]) + b_ref[...]
    h = jnp.maximum(pre, 0.0)
    h = jnp.dot(h, w_ref[...], preferred_element_type=jnp.float32)
    out_ref[...] = h * dis[:, None]


def _fin_body(agg_ref, hs_ref, deg_ref, b_ref, wl_ref, bl_ref, out_ref):
    dis = _dis_from_deg(deg_ref[...])
    pre = dis[:, None] * (agg_ref[0] + agg_ref[1] + hs_ref[...]) + b_ref[...]
    h = jnp.maximum(pre, 0.0)
    out_ref[...] = (
        jnp.dot(h, wl_ref[...], preferred_element_type=jnp.float32)
        + bl_ref[...]
    )


_BLK = 128
_GRID = NP // _BLK

_spec_rows = pl.BlockSpec((_BLK, D), lambda i: (i, 0))
_spec_w = pl.BlockSpec((D, D), lambda i: (0, 0))
_spec_agg = pl.BlockSpec((NC, _BLK, D), lambda i: (0, i, 0))
_spec_b = pl.BlockSpec((1, D), lambda i: (0, 0))

_b1_call = pl.pallas_call(
    _b1_body,
    grid=(_GRID,),
    in_specs=[_spec_rows, _spec_w, _spec_agg],
    out_specs=_spec_rows,
    out_shape=jax.ShapeDtypeStruct((NP, D), jnp.float32),
)

_bn_call = pl.pallas_call(
    _bn_body,
    grid=(_GRID,),
    in_specs=[_spec_agg, _spec_rows, _spec_agg, _spec_b, _spec_w],
    out_specs=_spec_rows,
    out_shape=jax.ShapeDtypeStruct((NP, D), jnp.float32),
)

_fin_call = pl.pallas_call(
    _fin_body,
    grid=(1,),
    in_specs=[
        pl.BlockSpec((NC, 8, D), lambda i: (0, 0, 0)),
        pl.BlockSpec((8, D), lambda i: (0, 0)),
        pl.BlockSpec((NC, 8, D), lambda i: (0, 0, 0)),
        _spec_b,
        _spec_w,
        _spec_b,
    ],
    out_specs=pl.BlockSpec((8, D), lambda i: (0, 0)),
    out_shape=jax.ShapeDtypeStruct((8, D), jnp.float32),
)


def kernel(x, edge_index, W1, b1, W2, b2, W3, b3, Wl, bl):
    x_pad = jnp.pad(x, ((0, NP - N), (0, 0)))
    src = edge_index[0].reshape(NW, EPW)
    dst = edge_index[1].reshape(NW, EPW)
    pad = NP - EPW
    src_pad = jnp.pad(src, ((0, 0), (0, pad)),
                      constant_values=PAD_SRC).reshape(NW, CH, CS)
    dst_pad = jnp.pad(dst, ((0, 0), (0, pad)),
                      constant_values=PAD_DST).reshape(NW, CH, CS)
    comb = jnp.stack([src_pad, dst_pad], axis=2)
    comb = comb.reshape(NW, _NB, _BC * 2, CS)
    b1r = b1.reshape(1, D)
    b2r = b2.reshape(1, D)
    b3r = b3.reshape(1, D)
    wl_pad = jnp.pad(Wl, ((0, 0), (0, D - Wl.shape[1])))
    bl_pad = jnp.pad(bl, ((0, D - bl.shape[0]),)).reshape(1, D)

    ones_table = jnp.ones((NP, D), jnp.float32)
    deg = _edge_kernel(ones_table, comb)

    hs1 = _b1_call(x_pad, W1, deg)
    agg1 = _edge_kernel(hs1, comb)
    hs2 = _bn_call(agg1, hs1, deg, b1r, W2)
    agg2 = _edge_kernel(hs2, comb)
    hs3 = _bn_call(agg2, hs2, deg, b2r, W3)
    agg3 = _edge_kernel(hs3, comb)
    out = _fin_call(agg3, hs3, deg, b3r, wl_pad, bl_pad)
    return out[:5, :3]


# speedup vs baseline: 11.3779x; 1.5084x over previous
import jax
import jax.numpy as jnp
from jax import lax
from jax.experimental import pallas as pl
from jax.experimental.pallas import tpu as pltpu
from jax.experimental.pallas import tpu_sc as plsc

N = 10000
E = 320000
D = 128
NP = 10240
NC = 2
NS = 16
NW = NC * NS
EPW = E // NW
CS = 128
CH = NP // CS
RPT = NP // NS
KO = RPT // CS
PAD_SRC = N
PAD_DST = NP - 1

_mesh = plsc.VectorSubcoreMesh(core_axis_name="c", subcore_axis_name="s",
                               num_cores=NC, num_subcores=NS)


def _row_fill(ref, n, valfn):
    def f(i, _):
        for j16 in range(D // 16):
            ref[i, pl.ds(j16 * 16, 16)] = valfn(i, j16)
        return 0
    lax.fori_loop(0, n, f, 0)


def _fill_own_idx(idx_ref, s, k):
    base = s * RPT + k * CS
    for j16 in range(CS // 16):
        idx_ref[pl.ds(j16 * 16, 16)] = (base + j16 * 16
                                        + lax.iota(jnp.int32, 16))


_BC = 10
_NB = CH // _BC


def _edge_body(hs_hbm, idx_hbm, flags_hbm, hits_hbm, agg_out, rows0, rows1,
               ibuf, didx, fbuf, flv, acc, g0, g1, s0, s1):
    c = lax.axis_index("c")
    s = lax.axis_index("s")
    w = c * NS + s
    _row_fill(rows0, CS, lambda i, j16: jnp.zeros((16,), jnp.float32))
    for k in range(KO):
        _fill_own_idx(didx, s, k)
        pltpu.async_copy(rows0, acc.at[didx], s0).wait()
    plsc.subcore_barrier()

    pltpu.sync_copy(flags_hbm, fbuf)
    ones_mode = fbuf[1, :][0]

    bufs = (rows0, rows1)
    gsems = (g0, g1)
    ssems = (s0, s1)

    @pl.when(ones_mode == 0)
    def _():
        def body(b, _):
            pltpu.sync_copy(idx_hbm.at[w, b], ibuf)
            pltpu.sync_copy(hits_hbm.at[w, b], flv)
            for k in range(_BC):
                flag = flv[pl.ds((k // 16) * 16, 16)][k % 16]
                @pl.when(flag != 0)
                def _():
                    pltpu.async_copy(hs_hbm.at[ibuf.at[2 * k]], rows0,
                                     g0).wait()
                    pltpu.async_copy(rows0, acc.at[ibuf.at[2 * k + 1]],
                                     s0, add=True).wait()
            return 0
        lax.fori_loop(0, _NB, body, 0)

    @pl.when(ones_mode == 1)
    def _():
        _row_fill(rows0, CS, lambda i, j16: jnp.ones((16,), jnp.float32))
        _row_fill(rows1, CS, lambda i, j16: jnp.ones((16,), jnp.float32))

        def body(b, _):
            pltpu.sync_copy(idx_hbm.at[w, b], ibuf)
            cs_pair = [None, None]
            for k in range(_BC):
                p = k & 1
                if cs_pair[p] is not None:
                    cs_pair[p].wait()
                cs_pair[p] = pltpu.async_copy(
                    bufs[p], acc.at[ibuf.at[2 * k + 1]], ssems[p],
                    add=True)
            cs_pair[0].wait()
            cs_pair[1].wait()
            return 0
        lax.fori_loop(0, _NB, body, 0)

    plsc.subcore_barrier()
    for k in range(KO):
        _fill_own_idx(didx, s, k)
        pltpu.async_copy(acc.at[didx], rows0, g0).wait()
        pltpu.sync_copy(rows0, agg_out.at[c, pl.ds(s * RPT + k * CS, CS)])


_edge_kernel = pl.kernel(
    _edge_body,
    out_type=jax.ShapeDtypeStruct((NC, NP, D), jnp.float32),
    mesh=_mesh,
    scratch_types=[
        pltpu.VMEM((CS, D), jnp.float32),
        pltpu.VMEM((CS, D), jnp.float32),
        pltpu.VMEM((2 * _BC, CS), jnp.int32),
        pltpu.VMEM((CS,), jnp.int32),
        pltpu.VMEM((2, 16), jnp.int32),
        pltpu.VMEM((CS,), jnp.int32),
        pltpu.VMEM_SHARED((NP, D), jnp.float32),
        pltpu.SemaphoreType.DMA,
        pltpu.SemaphoreType.DMA,
        pltpu.SemaphoreType.DMA,
        pltpu.SemaphoreType.DMA,
    ],
)


def _dis_from_deg(deg_blk):
    counts = deg_blk[0, :, 0] + deg_blk[1, :, 0] + 1.0
    return lax.rsqrt(counts)


def _b1_body(x_ref, w_ref, deg_ref, out_ref):
    dis = _dis_from_deg(deg_ref[...])
    h = jnp.dot(x_ref[...], w_ref[...], preferred_element_type=jnp.float32)
    out_ref[...] = h * dis[:, None]


def _bn_body(agg_ref, hs_ref, deg_ref, b_ref, w_ref, out_ref):
    dis = _dis_from_deg(deg_ref[...])
    pre = dis[:, None] * (agg_ref[0] + agg_ref[1] + hs_ref[...]) + b_ref[...]
    h = jnp.maximum(pre, 0.0)
    h = jnp.dot(h, w_ref[...], preferred_element_type=jnp.float32)
    out_ref[...] = h * dis[:, None]


def _fin_body(agg_ref, hs_ref, deg_ref, b_ref, wl_ref, bl_ref, out_ref):
    dis = _dis_from_deg(deg_ref[...])
    pre = dis[:, None] * (agg_ref[0] + agg_ref[1] + hs_ref[...]) + b_ref[...]
    h = jnp.maximum(pre, 0.0)
    out_ref[...] = (
        jnp.dot(h, wl_ref[...], preferred_element_type=jnp.float32)
        + bl_ref[...]
    )


_BLK = 128
_GRID = NP // _BLK

_spec_rows = pl.BlockSpec((_BLK, D), lambda i: (i, 0))
_spec_w = pl.BlockSpec((D, D), lambda i: (0, 0))
_spec_agg = pl.BlockSpec((NC, _BLK, D), lambda i: (0, i, 0))
_spec_b = pl.BlockSpec((1, D), lambda i: (0, 0))

_b1_call = pl.pallas_call(
    _b1_body,
    grid=(_GRID,),
    in_specs=[_spec_rows, _spec_w, _spec_agg],
    out_specs=_spec_rows,
    out_shape=jax.ShapeDtypeStruct((NP, D), jnp.float32),
)

_bn_call = pl.pallas_call(
    _bn_body,
    grid=(_GRID,),
    in_specs=[_spec_agg, _spec_rows, _spec_agg, _spec_b, _spec_w],
    out_specs=_spec_rows,
    out_shape=jax.ShapeDtypeStruct((NP, D), jnp.float32),
)

_fin_call = pl.pallas_call(
    _fin_body,
    grid=(1,),
    in_specs=[
        pl.BlockSpec((NC, 8, D), lambda i: (0, 0, 0)),
        pl.BlockSpec((8, D), lambda i: (0, 0)),
        pl.BlockSpec((NC, 8, D), lambda i: (0, 0, 0)),
        _spec_b,
        _spec_w,
        _spec_b,
    ],
    out_specs=pl.BlockSpec((8, D), lambda i: (0, 0)),
    out_shape=jax.ShapeDtypeStruct((8, D), jnp.float32),
)


def kernel(x, edge_index, W1, b1, W2, b2, W3, b3, Wl, bl):
    x_pad = jnp.pad(x, ((0, NP - N), (0, 0)))
    src = edge_index[0].reshape(NW, EPW)
    dst = edge_index[1].reshape(NW, EPW)
    pad = NP - EPW
    src_pad = jnp.pad(src, ((0, 0), (0, pad)),
                      constant_values=PAD_SRC).reshape(NW, CH, CS)
    dst_pad = jnp.pad(dst, ((0, 0), (0, pad)),
                      constant_values=PAD_DST).reshape(NW, CH, CS)
    comb = jnp.stack([src_pad, dst_pad], axis=2)
    comb = comb.reshape(NW, _NB, _BC * 2, CS)
    b1r = b1.reshape(1, D)
    b2r = b2.reshape(1, D)
    b3r = b3.reshape(1, D)
    wl_pad = jnp.pad(Wl, ((0, 0), (0, D - Wl.shape[1])))
    bl_pad = jnp.pad(bl, ((0, D - bl.shape[0]),)).reshape(1, D)

    def flags(ones_mode):
        return jnp.full((2, 16), ones_mode, jnp.int32)

    f_deg = flags(1)
    f_gat = flags(0)
    hits_all = jnp.ones((NW, _NB, CS), jnp.int32)
    hits_out = (dst_pad < 5).any(-1).astype(jnp.int32).reshape(NW, _NB, _BC)
    hits_out = jnp.pad(hits_out, ((0, 0), (0, 0), (0, CS - _BC)))

    deg = _edge_kernel(x_pad, comb, f_deg, hits_all)

    hs1 = _b1_call(x_pad, W1, deg)
    agg1 = _edge_kernel(hs1, comb, f_gat, hits_all)
    hs2 = _bn_call(agg1, hs1, deg, b1r, W2)
    agg2 = _edge_kernel(hs2, comb, f_gat, hits_all)
    hs3 = _bn_call(agg2, hs2, deg, b2r, W3)
    agg3 = _edge_kernel(hs3, comb, f_gat, hits_out)
    out = _fin_call(agg3, hs3, deg, b3r, wl_pad, bl_pad)
    return out[:5, :3]


# speedup vs baseline: 12.2109x; 1.0732x over previous
import jax
import jax.numpy as jnp
from jax import lax
from jax.experimental import pallas as pl
from jax.experimental.pallas import tpu as pltpu
from jax.experimental.pallas import tpu_sc as plsc

N = 10000
E = 320000
D = 128
NP = 10240
NC = 2
NS = 16
NW = NC * NS
EPW = E // NW
CS = 128
CH = NP // CS
RPT = NP // NS
KO = RPT // CS
PAD_SRC = N
PAD_DST = NP - 1

_mesh = plsc.VectorSubcoreMesh(core_axis_name="c", subcore_axis_name="s",
                               num_cores=NC, num_subcores=NS)


def _row_fill(ref, n, valfn):
    def f(i, _):
        for j16 in range(D // 16):
            ref[i, pl.ds(j16 * 16, 16)] = valfn(i, j16)
        return 0
    lax.fori_loop(0, n, f, 0)


def _fill_own_idx(idx_ref, s, k):
    base = s * RPT + k * CS
    for j16 in range(CS // 16):
        idx_ref[pl.ds(j16 * 16, 16)] = (base + j16 * 16
                                        + lax.iota(jnp.int32, 16))


_BC = 10
_NB = CH // _BC


def _edge_body(hs_hbm, idx_hbm, flags_hbm, hits_hbm, agg_out, rows0, rows1,
               ibuf, didx, fbuf, flv, acc, g0, g1, s0, s1):
    c = lax.axis_index("c")
    s = lax.axis_index("s")
    w = c * NS + s
    _row_fill(rows0, CS, lambda i, j16: jnp.zeros((16,), jnp.float32))
    for k in range(KO):
        _fill_own_idx(didx, s, k)
        pltpu.async_copy(rows0, acc.at[didx], s0).wait()
    plsc.subcore_barrier()

    pltpu.sync_copy(flags_hbm, fbuf)
    ones_mode = fbuf[1, :][0]

    bufs = (rows0, rows1)
    gsems = (g0, g1)
    ssems = (s0, s1)

    @pl.when(ones_mode == 0)
    def _():
        def body(b, _):
            pltpu.sync_copy(idx_hbm.at[w, b], ibuf)
            cg = pltpu.async_copy(hs_hbm.at[ibuf.at[0]], rows0, g0)
            cs_pair = [None, None]
            for k in range(_BC):
                p = k & 1
                cg.wait()
                cs_pair[p] = pltpu.async_copy(
                    bufs[p], acc.at[ibuf.at[2 * k + 1]], ssems[p],
                    add=True)
                if k + 1 < _BC:
                    if cs_pair[1 - p] is not None:
                        cs_pair[1 - p].wait()
                    cg = pltpu.async_copy(
                        hs_hbm.at[ibuf.at[2 * k + 2]], bufs[1 - p],
                        gsems[1 - p])
            cs_pair[0].wait()
            cs_pair[1].wait()
            return 0
        lax.fori_loop(0, _NB, body, 0)

    @pl.when(ones_mode == 2)
    def _():
        def body(b, _):
            pltpu.sync_copy(idx_hbm.at[w, b], ibuf)
            pltpu.sync_copy(hits_hbm.at[w, b], flv)
            for k in range(_BC):
                flag = flv[pl.ds((k // 16) * 16, 16)][k % 16]
                @pl.when(flag != 0)
                def _():
                    pltpu.async_copy(hs_hbm.at[ibuf.at[2 * k]], rows0,
                                     g0).wait()
                    pltpu.async_copy(rows0, acc.at[ibuf.at[2 * k + 1]],
                                     s0, add=True).wait()
            return 0
        lax.fori_loop(0, _NB, body, 0)

    @pl.when(ones_mode == 1)
    def _():
        _row_fill(rows0, CS, lambda i, j16: jnp.ones((16,), jnp.float32))
        _row_fill(rows1, CS, lambda i, j16: jnp.ones((16,), jnp.float32))

        def body(b, _):
            pltpu.sync_copy(idx_hbm.at[w, b], ibuf)
            cs_pair = [None, None]
            for k in range(_BC):
                p = k & 1
                if cs_pair[p] is not None:
                    cs_pair[p].wait()
                cs_pair[p] = pltpu.async_copy(
                    bufs[p], acc.at[ibuf.at[2 * k + 1]], ssems[p],
                    add=True)
            cs_pair[0].wait()
            cs_pair[1].wait()
            return 0
        lax.fori_loop(0, _NB, body, 0)

    plsc.subcore_barrier()
    for k in range(KO):
        _fill_own_idx(didx, s, k)
        pltpu.async_copy(acc.at[didx], rows0, g0).wait()
        pltpu.sync_copy(rows0, agg_out.at[c, pl.ds(s * RPT + k * CS, CS)])


_edge_kernel = pl.kernel(
    _edge_body,
    out_type=jax.ShapeDtypeStruct((NC, NP, D), jnp.float32),
    mesh=_mesh,
    scratch_types=[
        pltpu.VMEM((CS, D), jnp.float32),
        pltpu.VMEM((CS, D), jnp.float32),
        pltpu.VMEM((2 * _BC, CS), jnp.int32),
        pltpu.VMEM((CS,), jnp.int32),
        pltpu.VMEM((2, 16), jnp.int32),
        pltpu.VMEM((CS,), jnp.int32),
        pltpu.VMEM_SHARED((NP, D), jnp.float32),
        pltpu.SemaphoreType.DMA,
        pltpu.SemaphoreType.DMA,
        pltpu.SemaphoreType.DMA,
        pltpu.SemaphoreType.DMA,
    ],
)


def _dis_from_deg(deg_blk):
    counts = deg_blk[0, :, 0] + deg_blk[1, :, 0] + 1.0
    return lax.rsqrt(counts)


def _b1_body(x_ref, w_ref, deg_ref, out_ref):
    dis = _dis_from_deg(deg_ref[...])
    h = jnp.dot(x_ref[...], w_ref[...], preferred_element_type=jnp.float32)
    out_ref[...] = h * dis[:, None]


def _bn_body(agg_ref, hs_ref, deg_ref, b_ref, w_ref, out_ref):
    dis = _dis_from_deg(deg_ref[...])
    pre = dis[:, None] * (agg_ref[0] + agg_ref[1] + hs_ref[...]) + b_ref[...]
    h = jnp.maximum(pre, 0.0)
    h = jnp.dot(h, w_ref[...], preferred_element_type=jnp.float32)
    out_ref[...] = h * dis[:, None]


def _fin_body(agg_ref, hs_ref, deg_ref, b_ref, wl_ref, bl_ref, out_ref):
    dis = _dis_from_deg(deg_ref[...])
    pre = dis[:, None] * (agg_ref[0] + agg_ref[1] + hs_ref[...]) + b_ref[...]
    h = jnp.maximum(pre, 0.0)
    out_ref[...] = (
        jnp.dot(h, wl_ref[...], preferred_element_type=jnp.float32)
        + bl_ref[...]
    )


_BLK = 128
_GRID = NP // _BLK

_spec_rows = pl.BlockSpec((_BLK, D), lambda i: (i, 0))
_spec_w = pl.BlockSpec((D, D), lambda i: (0, 0))
_spec_agg = pl.BlockSpec((NC, _BLK, D), lambda i: (0, i, 0))
_spec_b = pl.BlockSpec((1, D), lambda i: (0, 0))

_b1_call = pl.pallas_call(
    _b1_body,
    grid=(_GRID,),
    in_specs=[_spec_rows, _spec_w, _spec_agg],
    out_specs=_spec_rows,
    out_shape=jax.ShapeDtypeStruct((NP, D), jnp.float32),
)

_bn_call = pl.pallas_call(
    _bn_body,
    grid=(_GRID,),
    in_specs=[_spec_agg, _spec_rows, _spec_agg, _spec_b, _spec_w],
    out_specs=_spec_rows,
    out_shape=jax.ShapeDtypeStruct((NP, D), jnp.float32),
)

_fin_call = pl.pallas_call(
    _fin_body,
    grid=(1,),
    in_specs=[
        pl.BlockSpec((NC, 8, D), lambda i: (0, 0, 0)),
        pl.BlockSpec((8, D), lambda i: (0, 0)),
        pl.BlockSpec((NC, 8, D), lambda i: (0, 0, 0)),
        _spec_b,
        _spec_w,
        _spec_b,
    ],
    out_specs=pl.BlockSpec((8, D), lambda i: (0, 0)),
    out_shape=jax.ShapeDtypeStruct((8, D), jnp.float32),
)


def kernel(x, edge_index, W1, b1, W2, b2, W3, b3, Wl, bl):
    x_pad = jnp.pad(x, ((0, NP - N), (0, 0)))
    src = edge_index[0].reshape(NW, EPW)
    dst = edge_index[1].reshape(NW, EPW)
    pad = NP - EPW
    src_pad = jnp.pad(src, ((0, 0), (0, pad)),
                      constant_values=PAD_SRC).reshape(NW, CH, CS)
    dst_pad = jnp.pad(dst, ((0, 0), (0, pad)),
                      constant_values=PAD_DST).reshape(NW, CH, CS)
    comb = jnp.stack([src_pad, dst_pad], axis=2)
    comb = comb.reshape(NW, _NB, _BC * 2, CS)
    b1r = b1.reshape(1, D)
    b2r = b2.reshape(1, D)
    b3r = b3.reshape(1, D)
    wl_pad = jnp.pad(Wl, ((0, 0), (0, D - Wl.shape[1])))
    bl_pad = jnp.pad(bl, ((0, D - bl.shape[0]),)).reshape(1, D)

    def flags(ones_mode):
        return jnp.full((2, 16), ones_mode, jnp.int32)

    f_deg = flags(1)
    f_gat = flags(0)
    f_prn = flags(2)
    hits_all = jnp.ones((NW, _NB, CS), jnp.int32)
    hits_out = (dst_pad < 5).any(-1).astype(jnp.int32).reshape(NW, _NB, _BC)
    hits_out = jnp.pad(hits_out, ((0, 0), (0, 0), (0, CS - _BC)))

    deg = _edge_kernel(x_pad, comb, f_deg, hits_all)

    hs1 = _b1_call(x_pad, W1, deg)
    agg1 = _edge_kernel(hs1, comb, f_gat, hits_all)
    hs2 = _bn_call(agg1, hs1, deg, b1r, W2)
    agg2 = _edge_kernel(hs2, comb, f_gat, hits_all)
    hs3 = _bn_call(agg2, hs2, deg, b2r, W3)
    agg3 = _edge_kernel(hs3, comb, f_prn, hits_out)
    out = _fin_call(agg3, hs3, deg, b3r, wl_pad, bl_pad)
    return out[:5, :3]
